# baseline (speedup 1.0000x reference)
import functools

import jax
import jax.numpy as jnp
from jax import lax
from jax.experimental import pallas as pl
from jax.experimental.pallas import tpu as pltpu
from jax.experimental.pallas import tpu_sc as plsc

N_ATOM = 2048
E_ATOM = 65536
N_MOTIF = 512
E_MOTIF = 2048
HIDDEN = 256
HEADS = 4
HD = HIDDEN // HEADS

NW = 32
CH = 64
PCOL = N_ATOM // 2
CWORDS = CH * PCOL
EBLK = 4096
MCH = N_MOTIF // NW

RB = 512
NRB = N_ATOM // RB

_PREC = jax.lax.Precision.DEFAULT


@functools.cache
def _sc_atom_kernel():
    mesh = plsc.VectorSubcoreMesh(core_axis_name="c", subcore_axis_name="s")
    return pl.kernel(
        _sc_atom_body,
        mesh=mesh,
        compiler_params=pltpu.CompilerParams(needs_layout_passes=False),
        out_type=jax.ShapeDtypeStruct((N_ATOM, PCOL), jnp.int32),
        scratch_types=[
            pltpu.VMEM((CH, PCOL), jnp.int32),
            pltpu.VMEM((EBLK,), jnp.int32),
            pltpu.VMEM((EBLK,), jnp.int32),
            pltpu.VMEM((EBLK,), jnp.int32),
            pltpu.VMEM((EBLK,), jnp.int32),
            pltpu.SemaphoreType.DMA,
            pltpu.SemaphoreType.DMA,
        ],
    )


@functools.cache
def _sc_motif_kernel():
    mesh = plsc.VectorSubcoreMesh(core_axis_name="c", subcore_axis_name="s")
    return pl.kernel(
        _sc_motif_body,
        mesh=mesh,
        compiler_params=pltpu.CompilerParams(needs_layout_passes=False),
        out_type=jax.ShapeDtypeStruct((N_MOTIF, N_MOTIF), jnp.float32),
        scratch_types=[
            pltpu.VMEM((MCH, N_MOTIF), jnp.float32),
            pltpu.VMEM((E_MOTIF,), jnp.int32),
        ],
    )


def _sc_atom_body(flat_hbm, val_hbm, a_out,
                  buf, fbuf0, fbuf1, vbuf0, vbuf1, sem0, sem1):
    wid = lax.axis_index("s") * 2 + lax.axis_index("c")
    izeros = jnp.zeros((16,), jnp.int32)
    fbase = wid * CWORDS
    fbufs, vbufs, sems = (fbuf0, fbuf1), (vbuf0, vbuf1), (sem0, sem1)
    nblk = E_ATOM // EBLK

    def _start(blk):
        slot = blk % 2
        return (
            pltpu.async_copy(flat_hbm.at[pl.ds(blk * EBLK, EBLK)],
                             fbufs[slot], sems[slot]),
            pltpu.async_copy(val_hbm.at[pl.ds(blk * EBLK, EBLK)],
                             vbufs[slot], sems[slot]),
        )

    pending = _start(0)

    def zbody(i, _):
        r = i >> 3
        cb = (i & 7) * 128
        for k in range(8):
            buf[r, pl.ds(cb + k * 16, 16)] = izeros
        return _
    lax.fori_loop(0, CWORDS // 128, zbody, 0)

    for blk in range(nblk):
        slot = blk % 2
        for h in pending:
            h.wait()
        if blk + 1 < nblk:
            pending = _start(blk + 1)
        fbuf, vbuf = fbufs[slot], vbufs[slot]

        def ebody(i, _):
            for k in range(8):
                off = i * 128 + k * 16
                rel = fbuf[pl.ds(off, 16)] - fbase
                m = rel.astype(jnp.uint32) < CWORDS
                plsc.addupdate_scatter(
                    buf, [rel >> 10, rel & (PCOL - 1)],
                    vbuf[pl.ds(off, 16)], mask=m)
            return _
        lax.fori_loop(0, EBLK // 128, ebody, 0)

    pltpu.sync_copy(buf, a_out.at[pl.ds(wid * CH, CH)])


def _sc_motif_body(mflat_hbm, am_out, mbuf, mfbuf):
    wid = lax.axis_index("s") * 2 + lax.axis_index("c")
    ones = jnp.ones((16,), jnp.float32)
    zeros = jnp.zeros((16,), jnp.float32)
    mwords = MCH * N_MOTIF
    mfbase = wid * mwords
    pltpu.sync_copy(mflat_hbm, mfbuf)

    def mzbody(i, _):
        r = i >> 2
        cb = (i & 3) * 128
        for k in range(8):
            mbuf[r, pl.ds(cb + k * 16, 16)] = zeros
        return _
    lax.fori_loop(0, mwords // 128, mzbody, 0)

    def mebody(i, _):
        for k in range(4):
            rel = mfbuf[pl.ds(i * 64 + k * 16, 16)] - mfbase
            m = rel.astype(jnp.uint32) < mwords
            plsc.addupdate_scatter(mbuf, [rel >> 9, rel & (N_MOTIF - 1)],
                                   ones, mask=m)
        return _
    lax.fori_loop(0, E_MOTIF // 64, mebody, 0)

    pltpu.sync_copy(mbuf, am_out.at[pl.ds(wid * MCH, MCH)])


def _dot(a, b, ca=1, cb=0):
    return lax.dot_general(a, b, (((ca,), (cb,)), ((), ())), precision=_PREC)


def _full(shape):
    return pl.BlockSpec(shape, lambda *_: (0,) * len(shape))


def _flat_body(e_ref, me_ref, f_ref, v_ref, mf_ref):
    src, dst = e_ref[0], e_ref[1]
    f_ref[...] = dst * PCOL + (src & (PCOL - 1))
    v_ref[...] = 1 << ((src >> 10) << 4)
    mf_ref[...] = me_ref[1] * N_MOTIF + me_ref[0]


def _flat_ids(edge_index, motif_edge_index):
    f, v, mf = pl.pallas_call(
        _flat_body,
        out_shape=[
            jax.ShapeDtypeStruct((E_ATOM // 128, 128), jnp.int32),
            jax.ShapeDtypeStruct((E_ATOM // 128, 128), jnp.int32),
            jax.ShapeDtypeStruct((E_MOTIF // 128, 128), jnp.int32),
        ],
    )(edge_index.reshape(2, E_ATOM // 128, 128),
      motif_edge_index.reshape(2, E_MOTIF // 128, 128))
    return f.reshape(E_ATOM), v.reshape(E_ATOM), mf.reshape(E_MOTIF)


def _gcn_pre_body(a_ref, xf_ref, aw_ref, ab_ref, w0_ref,
                  abf_ref, dinv_ref, hd_ref, hdb_ref):
    ap = a_ref[...]
    low = ap & 0xFFFF
    high = ap >> 16
    abf_ref[:, :PCOL] = low.astype(jnp.bfloat16)
    abf_ref[:, PCOL:] = high.astype(jnp.bfloat16)
    deg = (jnp.sum(low, axis=1, keepdims=True)
           + jnp.sum(high, axis=1, keepdims=True)).astype(jnp.float32) + 1.0
    dinv = lax.rsqrt(deg)
    dinv_ref[...] = dinv
    x0 = _dot(xf_ref[...], aw_ref[...]) + ab_ref[...]
    hd = dinv * _dot(x0, w0_ref[...])
    hd_ref[...] = hd
    hdb_ref[...] = hd.astype(jnp.bfloat16)


def _gcn_pre(A, atom_f, aw, ab, w0):
    return pl.pallas_call(
        _gcn_pre_body,
        grid=(NRB,),
        in_specs=[pl.BlockSpec((RB, PCOL), lambda i: (i, 0)),
                  pl.BlockSpec((RB, 128), lambda i: (i, 0)),
                  _full((128, HIDDEN)), _full((1, HIDDEN)),
                  _full((HIDDEN, HIDDEN))],
        out_specs=[pl.BlockSpec((RB, N_ATOM), lambda i: (i, 0)),
                   pl.BlockSpec((RB, 1), lambda i: (i, 0)),
                   pl.BlockSpec((RB, HIDDEN), lambda i: (i, 0)),
                   pl.BlockSpec((RB, HIDDEN), lambda i: (i, 0))],
        out_shape=[jax.ShapeDtypeStruct((N_ATOM, N_ATOM), jnp.bfloat16),
                   jax.ShapeDtypeStruct((N_ATOM, 1), jnp.float32),
                   jax.ShapeDtypeStruct((N_ATOM, HIDDEN), jnp.float32),
                   jax.ShapeDtypeStruct((N_ATOM, HIDDEN), jnp.bfloat16)],
    )(A, atom_f, aw, ab.reshape(1, HIDDEN), w0)


def _gcn_agg(abf_ref, hdbf_ref, hd_ref):
    t = lax.dot_general(abf_ref[...], hdbf_ref[...],
                        (((1,), (0,)), ((), ())),
                        preferred_element_type=jnp.float32)
    return t + hd_ref[...]


def _gcn_fused_mid(abf_ref, hdbf_ref, hd_ref, dinv_ref, b_ref,
                   wn_ref, o_ref, ob_ref):
    t = _gcn_agg(abf_ref, hdbf_ref, hd_ref)
    x = jnp.maximum(dinv_ref[...] * t + b_ref[...], 0.0)
    hd = dinv_ref[...] * _dot(x, wn_ref[...])
    o_ref[...] = hd
    ob_ref[...] = hd.astype(jnp.bfloat16)


def _gcn_fused_last(abf_ref, hdbf_ref, hd_ref, dinv_ref, b_ref, o_ref):
    t = _gcn_agg(abf_ref, hdbf_ref, hd_ref)
    o_ref[...] = jnp.maximum(dinv_ref[...] * t + b_ref[...], 0.0)


def _gcn_fused(Abf, hd, hdb, dinv, b, w_next):
    specs = [pl.BlockSpec((RB, N_ATOM), lambda i: (i, 0)),
             _full((N_ATOM, HIDDEN)),
             pl.BlockSpec((RB, HIDDEN), lambda i: (i, 0)),
             pl.BlockSpec((RB, 1), lambda i: (i, 0)),
             _full((1, HIDDEN))]
    args = [Abf, hdb, hd, dinv, b.reshape(1, HIDDEN)]
    blk = pl.BlockSpec((RB, HIDDEN), lambda i: (i, 0))
    if w_next is None:
        return pl.pallas_call(
            _gcn_fused_last,
            grid=(NRB,),
            in_specs=specs,
            out_specs=blk,
            out_shape=jax.ShapeDtypeStruct((N_ATOM, HIDDEN), jnp.float32),
        )(*args)
    specs.append(_full((HIDDEN, HIDDEN)))
    args.append(w_next)
    return pl.pallas_call(
        _gcn_fused_mid,
        grid=(NRB,),
        in_specs=specs,
        out_specs=[blk, blk],
        out_shape=[jax.ShapeDtypeStruct((N_ATOM, HIDDEN), jnp.float32),
                   jax.ShapeDtypeStruct((N_ATOM, HIDDEN), jnp.bfloat16)],
    )(*args)


def _softmax_exp(s):
    return jnp.exp((s - jnp.max(s, axis=1, keepdims=True))
                   .astype(jnp.bfloat16))


def _motif_body(am_ref, mf_ref, mw_ref, mb_ref,
                w1_ref, b1_ref, w2_ref, b2_ref,
                wqkv_ref, bqkv_ref, o_ref):
    m = _dot(mf_ref[...], mw_ref[...]) + mb_ref[...]
    Am = am_ref[...]
    for i in range(3):
        h = m + _dot(Am, m)
        h1 = jnp.maximum(_dot(h, w1_ref[i]) + b1_ref[i][None, :], 0.0)
        m = jnp.maximum(_dot(h1, w2_ref[i]) + b2_ref[i][None, :], 0.0)
    qkv = _dot(m, wqkv_ref[...], 1, 1) + bqkv_ref[...]
    outs = []
    scale = 1.0 / float(HD) ** 0.5
    for h in range(HEADS):
        q = qkv[:, h * HD:(h + 1) * HD] * scale
        k = qkv[:, HIDDEN + h * HD:HIDDEN + (h + 1) * HD]
        v = qkv[:, 2 * HIDDEN + h * HD:2 * HIDDEN + (h + 1) * HD]
        s = lax.dot_general(q, k, (((1,), (1,)), ((), ())),
                            precision=_PREC)
        e = _softmax_exp(s)
        vx = jnp.concatenate(
            [v, jnp.ones((N_MOTIF, 16), jnp.float32)], axis=1)
        ox = lax.dot_general(e, vx.astype(jnp.bfloat16),
                             (((1,), (0,)), ((), ())),
                             preferred_element_type=jnp.float32)
        o = ox[:, :HD] * (1.0 / ox[:, HD:HD + 1])
        outs.append(jnp.sum(o, axis=0, keepdims=True) * (1.0 / N_MOTIF))
    o_ref[...] = jnp.concatenate(outs, axis=1)


def _motif_pooled(Am, motif_f, mw, mb, w1, b1, w2, b2, wqkv, bqkv):
    return pl.pallas_call(
        _motif_body,
        out_shape=jax.ShapeDtypeStruct((1, HIDDEN), jnp.float32),
    )(Am, motif_f, mw, mb.reshape(1, HIDDEN), w1, b1, w2, b2,
      wqkv, bqkv.reshape(1, 3 * HIDDEN))


def _attn_pool_body(xb_ref, xf_ref, wq_ref, wk_ref, wv_ref,
                    bq_ref, bk_ref, bv_ref, o_ref, k_s, v_s):
    j = pl.program_id(1)
    L = xf_ref.shape[0]

    @pl.when(j == 0)
    def _():
        k_s[...] = _dot(xf_ref[...], wk_ref[...], 1, 1) + bk_ref[0]
        v = _dot(xf_ref[...], wv_ref[...], 1, 1) + bv_ref[0]
        v_s[:, :HD] = v.astype(jnp.bfloat16)
        v_s[:, HD:HD + 16] = jnp.ones((L, 16), jnp.bfloat16)

    q = (_dot(xb_ref[...], wq_ref[...], 1, 1) + bq_ref[0]) \
        * (1.0 / float(HD) ** 0.5)
    s = lax.dot_general(q, k_s[...], (((1,), (1,)), ((), ())),
                        precision=_PREC)
    e = _softmax_exp(s)
    ox = lax.dot_general(e, v_s[...], (((1,), (0,)), ((), ())),
                         preferred_element_type=jnp.float32)
    o = ox[:, :HD] * (1.0 / ox[:, HD:HD + 1])
    colsum = jnp.sum(o, axis=0, keepdims=True) * (1.0 / float(L))

    @pl.when(j == 0)
    def _():
        o_ref[...] = jnp.zeros_like(o_ref)

    o_ref[0] += colsum


def _attn_pool(x, wqkv, bqkv):
    L = x.shape[0]
    b3 = bqkv.reshape(3 * HEADS, 1, HD)
    return pl.pallas_call(
        _attn_pool_body,
        grid=(HEADS, L // RB),
        in_specs=[
            pl.BlockSpec((RB, HIDDEN), lambda h, j: (j, 0)),
            _full((L, HIDDEN)),
            pl.BlockSpec((HD, HIDDEN), lambda h, j: (h, 0)),
            pl.BlockSpec((HD, HIDDEN), lambda h, j: (HEADS + h, 0)),
            pl.BlockSpec((HD, HIDDEN), lambda h, j: (2 * HEADS + h, 0)),
            pl.BlockSpec((1, 1, HD), lambda h, j: (h, 0, 0)),
            pl.BlockSpec((1, 1, HD), lambda h, j: (HEADS + h, 0, 0)),
            pl.BlockSpec((1, 1, HD), lambda h, j: (2 * HEADS + h, 0, 0)),
        ],
        out_specs=pl.BlockSpec((1, 1, HD), lambda h, j: (h, 0, 0)),
        out_shape=jax.ShapeDtypeStruct((HEADS, 1, HD), jnp.float32),
        scratch_shapes=[pltpu.VMEM((L, HD), jnp.float32),
                        pltpu.VMEM((L, HD + 16), jnp.bfloat16)],
    )(x, x, wqkv, wqkv, wqkv, b3, b3, b3)


def _final_body(am_ref, mm_ref, wo_ref, bo_ref,
                w1_ref, b1_ref, w2_ref, b2_ref, o_ref):
    ag = _dot(am_ref[...], wo_ref[...], 1, 1) + bo_ref[...]
    mg = _dot(mm_ref[...], wo_ref[...], 1, 1) + bo_ref[...]
    c = jnp.concatenate([ag, mg], axis=1)
    h = jnp.maximum(_dot(c, w1_ref[...]) + b1_ref[...], 0.0)
    o_ref[...] = _dot(h, w2_ref[...]) + b2_ref[...]


def _final(atom_mean, motif_mean, wo, bo, w1, b1, w2, b2):
    return pl.pallas_call(
        _final_body,
        out_shape=jax.ShapeDtypeStruct((1, 128), jnp.float32),
    )(atom_mean, motif_mean, wo, bo.reshape(1, HIDDEN),
      w1, b1.reshape(1, HIDDEN), w2, b2.reshape(1, 128))


def _build_adjacency(edge_index, motif_edge_index):
    flat, vals, mflat = _flat_ids(edge_index, motif_edge_index)
    Am = _sc_motif_kernel()(mflat)
    A = _sc_atom_kernel()(flat, vals)
    return A, Am


def kernel(atom_features, bond_features, motif_features, params,
           edge_index, motif_edge_index):
    del bond_features
    p = params
    A, Am = _build_adjacency(edge_index, motif_edge_index)

    mm = _motif_pooled(Am, motif_features, p['motif_W'], p['motif_b'],
                       p['gin_W1'], p['gin_b1'], p['gin_W2'], p['gin_b2'],
                       p['attn_Wqkv'], p['attn_bqkv'])

    Abf, dinv, hd, hdb = _gcn_pre(A, atom_features, p['atom_W'],
                                  p['atom_b'], p['gcn_W'][0])
    hd, hdb = _gcn_fused(Abf, hd, hdb, dinv, p['gcn_b'][0], p['gcn_W'][1])
    hd, hdb = _gcn_fused(Abf, hd, hdb, dinv, p['gcn_b'][1], p['gcn_W'][2])
    x = _gcn_fused(Abf, hd, hdb, dinv, p['gcn_b'][2], None)

    am = _attn_pool(x, p['attn_Wqkv'], p['attn_bqkv']).reshape(1, HIDDEN)

    latent = _final(am, mm, p['attn_Wo'], p['attn_bo'],
                    p['proj_W1'], p['proj_b1'], p['proj_W2'], p['proj_b2'])
    return latent.reshape(128)
